# Initial kernel scaffold; baseline (speedup 1.0000x reference)
import jax
import jax.numpy as jnp
from jax.experimental import pallas as pl


def kernel(x, edge_index, edge_attr, W, b):
    raise NotImplementedError("write your pallas kernel here")


# speedup vs baseline: 13.1917x; 13.1917x over previous
import functools

import jax
import jax.numpy as jnp
from jax import lax
from jax.experimental import pallas as pl
from jax.experimental.pallas import tpu as pltpu
from jax.experimental.pallas import tpu_sc as plsc

NC, NS, L = 2, 16, 16
NW = NC * NS

_N = 10000
_E = 320000
_D = 128
_NPAD = 10240
_ACCR = 10112
_TRASH = 10048
_G = 64
_EPW = _E // NW
_NCHUNK = (_EPW + _G - 1) // _G
_LAST = _EPW - (_NCHUNK - 1) * _G
_EPAD = _NCHUNK * _G
_EPS = _E // NS

_mesh = plsc.VectorSubcoreMesh(
    core_axis_name="c", subcore_axis_name="s", num_cores=NC, num_subcores=NS
)
_sc_params = pltpu.CompilerParams(
    needs_layout_passes=False, use_tc_tiling_on_sc=False
)


def _fast_rsqrt(m):
    half = m * 0.5
    i = plsc.bitcast(m, jnp.int32)
    i = jnp.int32(0x5F3759DF) - lax.shift_right_arithmetic(i, 1)
    y = plsc.bitcast(i, jnp.float32)
    y = y * (1.5 - half * y * y)
    y = y * (1.5 - half * y * y)
    y = y * (1.5 - half * y * y)
    return y


@functools.partial(
    pl.kernel,
    out_type=jax.ShapeDtypeStruct((_NPAD,), jnp.float32),
    mesh=_mesh,
    compiler_params=_sc_params,
    scratch_types=[
        pltpu.VMEM((_EPS,), jnp.int32),
        pltpu.VMEM((_EPS,), jnp.float32),
        pltpu.VMEM((_NPAD,), jnp.float32),
        pltpu.VMEM_SHARED((NS * _NPAD,), jnp.float32),
        pltpu.VMEM((NS * (_NPAD // NW),), jnp.float32),
        pltpu.VMEM((_NPAD // NW,), jnp.float32),
    ],
)
def _sc_dinv(dst_hbm, attr_hbm, dinv_hbm, dst_v, attr_v, deg_v, slab_sh,
             sbuf_v, dchunk_v):
    c = lax.axis_index("c")
    s = lax.axis_index("s")
    wid = c * NS + s
    nb = _NPAD // NW

    zero16 = jnp.zeros((L,), jnp.float32)

    def zbody(i, carry):
        deg_v[pl.ds(i * L, L)] = zero16
        return carry

    lax.fori_loop(0, _NPAD // L, zbody, 0)

    base = s * _EPS
    pltpu.sync_copy(dst_hbm.at[pl.ds(base, _EPS)], dst_v)
    pltpu.sync_copy(attr_hbm.at[pl.ds(base, _EPS)], attr_v)

    def ebody(i, carry):
        dv = dst_v[pl.ds(i * L, L)]
        av = attr_v[pl.ds(i * L, L)]
        plsc.addupdate_scatter(deg_v, [dv], av)
        return carry

    lax.fori_loop(0, _EPS // L, ebody, 0)

    pltpu.sync_copy(deg_v, slab_sh.at[pl.ds(s * _NPAD, _NPAD)])
    plsc.subcore_barrier()

    for r in range(NS):
        pltpu.sync_copy(
            slab_sh.at[pl.ds(r * _NPAD + wid * nb, nb)],
            sbuf_v.at[pl.ds(r * nb, nb)],
        )

    def rbody(v, carry):
        acc = sbuf_v[pl.ds(v * L, L)]
        for r in range(1, NS):
            acc = acc + sbuf_v[pl.ds(r * nb + v * L, L)]
        y = _fast_rsqrt(jnp.maximum(acc, 1e-12))
        dchunk_v[pl.ds(v * L, L)] = jnp.where(acc > 0.0, y, 0.0)
        return carry

    lax.fori_loop(0, nb // L, rbody, 0)
    pltpu.sync_copy(dchunk_v, dinv_hbm.at[pl.ds(wid * nb, nb)])


@functools.partial(
    pl.kernel,
    out_type=jax.ShapeDtypeStruct((NC, _ACCR, _D), jnp.float32),
    mesh=_mesh,
    compiler_params=_sc_params,
    scratch_types=[
        pltpu.VMEM((_NPAD,), jnp.float32),
        pltpu.VMEM((_EPAD,), jnp.int32),
        pltpu.VMEM((_EPAD,), jnp.float32),
        pltpu.VMEM((_NCHUNK, _G), jnp.int32),
        pltpu.VMEM((_G, _D), jnp.float32),
        pltpu.VMEM_SHARED((_ACCR, _D), jnp.float32),
    ],
)
def _sc_agg(h_hbm, src_hbm, dst_hbm, attr_hbm, dinv_hbm, part_hbm,
            dinv_v, src_v, norm_v, dst2d_v, rows_v, acc_sh):
    c = lax.axis_index("c")
    s = lax.axis_index("s")
    wid = c * NS + s
    rows_per_tile = _ACCR // NS

    zero16 = jnp.zeros((L,), jnp.float32)

    def zb(i, carry):
        for q in range(_D // L):
            rows_v[i, pl.ds(q * L, L)] = zero16
        return carry

    lax.fori_loop(0, _G, zb, 0)

    nz_full = rows_per_tile // _G
    def zacc(j, carry):
        pltpu.sync_copy(rows_v, acc_sh.at[pl.ds(s * rows_per_tile + j * _G, _G)])
        return carry

    lax.fori_loop(0, nz_full, zacc, 0)
    pltpu.sync_copy(
        rows_v, acc_sh.at[pl.ds(s * rows_per_tile + rows_per_tile - _G, _G)]
    )

    base = wid * _EPW
    pltpu.sync_copy(dinv_hbm.at[pl.ds(0, _NPAD)], dinv_v)
    pltpu.sync_copy(src_hbm.at[pl.ds(base, _EPW)], src_v.at[pl.ds(0, _EPW)])
    pltpu.sync_copy(attr_hbm.at[pl.ds(base, _EPW)], norm_v.at[pl.ds(0, _EPW)])

    lane = lax.iota(jnp.int32, L)

    def nbody(j, carry):
        @pl.when(j < _NCHUNK - 1)
        def _full():
            pltpu.sync_copy(dst_hbm.at[pl.ds(base + j * _G, _G)], dst2d_v.at[j])

        @pl.when(j == _NCHUNK - 1)
        def _tail():
            pltpu.sync_copy(
                dst_hbm.at[pl.ds(base + (_NCHUNK - 1) * _G, _LAST)],
                dst2d_v.at[j, pl.ds(0, _LAST)],
            )

        for k in range(_G // L):
            i = j * (_G // L) + k
            sv = src_v[pl.ds(i * L, L)]
            dv = dst2d_v[j, pl.ds(k * L, L)]
            av = norm_v[pl.ds(i * L, L)]
            valid = (i * L + lane) < _EPW
            sv = jnp.where(valid, sv, 0)
            dv = jnp.where(valid, dv, _TRASH)
            nv = plsc.load_gather(dinv_v, [sv]) * plsc.load_gather(dinv_v, [dv])
            nv = jnp.where(valid, nv * av, 0.0)
            src_v[pl.ds(i * L, L)] = sv
            norm_v[pl.ds(i * L, L)] = nv
            dst2d_v[j, pl.ds(k * L, L)] = dv
        return carry

    lax.fori_loop(0, _NCHUNK, nbody, 0)

    plsc.subcore_barrier()

    def mbody(j, carry):
        pltpu.sync_copy(h_hbm.at[src_v.at[pl.ds(j * _G, _G)]], rows_v)

        def sbody(e, c2):
            idx = jnp.zeros((L,), jnp.int32) + (j * _G + e)
            nsc = plsc.load_gather(norm_v, [idx])
            for q in range(_D // L):
                rows_v[e, pl.ds(q * L, L)] = rows_v[e, pl.ds(q * L, L)] * nsc
            return c2

        lax.fori_loop(0, _G, sbody, 0)
        pltpu.sync_copy(rows_v, acc_sh.at[dst2d_v.at[j]], add=True)
        return carry

    lax.fori_loop(0, _NCHUNK, mbody, 0)

    plsc.subcore_barrier()
    pltpu.sync_copy(
        acc_sh.at[pl.ds(s * rows_per_tile, rows_per_tile)],
        part_hbm.at[c, pl.ds(s * rows_per_tile, rows_per_tile)],
    )


def _mm_body(x_ref, w_ref, o_ref):
    o_ref[...] = jnp.dot(
        x_ref[...], w_ref[...],
        preferred_element_type=jnp.float32,
        precision=lax.Precision.HIGHEST,
    )


def _comb_body(p0_ref, p1_ref, b_ref, o_ref):
    o_ref[...] = p0_ref[...] + p1_ref[...] + b_ref[...]


def kernel(x, edge_index, edge_attr, W, b):
    src = edge_index[0]
    dst = edge_index[1]

    h = pl.pallas_call(
        _mm_body,
        grid=(10,),
        in_specs=[
            pl.BlockSpec((_N // 10, _D), lambda i: (i, 0)),
            pl.BlockSpec((_D, _D), lambda i: (0, 0)),
        ],
        out_specs=pl.BlockSpec((_N // 10, _D), lambda i: (i, 0)),
        out_shape=jax.ShapeDtypeStruct((_N, _D), jnp.float32),
    )(x, W)

    dinv = _sc_dinv(dst, edge_attr)
    part = _sc_agg(h, src, dst, edge_attr, dinv)

    out = pl.pallas_call(
        _comb_body,
        grid=(10,),
        in_specs=[
            pl.BlockSpec((_N // 10, _D), lambda i: (i, 0)),
            pl.BlockSpec((_N // 10, _D), lambda i: (i, 0)),
            pl.BlockSpec((1, _D), lambda i: (0, 0)),
        ],
        out_specs=pl.BlockSpec((_N // 10, _D), lambda i: (i, 0)),
        out_shape=jax.ShapeDtypeStruct((_N, _D), jnp.float32),
    )(part[0, :_N], part[1, :_N], b.reshape(1, _D))
    return out


# speedup vs baseline: 20.4027x; 1.5466x over previous
import functools

import jax
import jax.numpy as jnp
from jax import lax
from jax.experimental import pallas as pl
from jax.experimental.pallas import tpu as pltpu
from jax.experimental.pallas import tpu_sc as plsc

NC, NS, L = 2, 16, 16
NW = NC * NS

_N = 10000
_E = 320000
_D = 128
_NPAD = 10240
_ACCR = 10112
_TRASH = 10048
_G = 48
_NBUF = 3
_EPW = _E // NW
_NCHUNK = 210
_EPAD = _NCHUNK * _G
_EPS = _E // NS

_mesh = plsc.VectorSubcoreMesh(
    core_axis_name="c", subcore_axis_name="s", num_cores=NC, num_subcores=NS
)
_sc_params = pltpu.CompilerParams(
    needs_layout_passes=False, use_tc_tiling_on_sc=False
)


def _fast_rsqrt(m):
    half = m * 0.5
    i = plsc.bitcast(m, jnp.int32)
    i = jnp.int32(0x5F3759DF) - lax.shift_right_arithmetic(i, 1)
    y = plsc.bitcast(i, jnp.float32)
    y = y * (1.5 - half * y * y)
    y = y * (1.5 - half * y * y)
    y = y * (1.5 - half * y * y)
    return y


@functools.partial(
    pl.kernel,
    out_type=jax.ShapeDtypeStruct((_E,), jnp.float32),
    mesh=_mesh,
    compiler_params=_sc_params,
    scratch_types=[
        pltpu.VMEM((_EPS,), jnp.int32),
        pltpu.VMEM((_EPS,), jnp.float32),
        pltpu.VMEM((_NPAD,), jnp.float32),
        pltpu.VMEM((NS * (_NPAD // NS),), jnp.float32),
        pltpu.VMEM((_NPAD // NS,), jnp.float32),
        pltpu.VMEM((_NPAD,), jnp.float32),
        pltpu.VMEM_SHARED((NS * _NPAD,), jnp.float32),
        pltpu.VMEM_SHARED((_NPAD,), jnp.float32),
    ],
)
def _sc_norm(src_hbm, dst_hbm, attr_hbm, norm_hbm, idx_v, attr_v, deg_v,
             sbuf_v, dchunk_v, dinv_v, slab_sh, dinv_sh):
    c = lax.axis_index("c")
    s = lax.axis_index("s")
    wid = c * NS + s
    nb = _NPAD // NS

    zero16 = jnp.zeros((L,), jnp.float32)

    def zbody(i, carry):
        deg_v[pl.ds(i * L, L)] = zero16
        return carry

    lax.fori_loop(0, _NPAD // L, zbody, 0)

    dbase = s * _EPS
    pltpu.sync_copy(dst_hbm.at[pl.ds(dbase, _EPS)], idx_v)
    pltpu.sync_copy(attr_hbm.at[pl.ds(dbase, _EPS)], attr_v)

    def ebody(i, carry):
        dv = idx_v[pl.ds(i * L, L)]
        av = attr_v[pl.ds(i * L, L)]
        plsc.addupdate_scatter(deg_v, [dv], av)
        return carry

    lax.fori_loop(0, _EPS // L, ebody, 0)

    pltpu.sync_copy(deg_v, slab_sh.at[pl.ds(s * _NPAD, _NPAD)])
    plsc.subcore_barrier()

    for r in range(NS):
        pltpu.sync_copy(
            slab_sh.at[pl.ds(r * _NPAD + s * nb, nb)],
            sbuf_v.at[pl.ds(r * nb, nb)],
        )

    def rbody(v, carry):
        acc = sbuf_v[pl.ds(v * L, L)]
        for r in range(1, NS):
            acc = acc + sbuf_v[pl.ds(r * nb + v * L, L)]
        y = _fast_rsqrt(jnp.maximum(acc, 1e-12))
        dchunk_v[pl.ds(v * L, L)] = jnp.where(acc > 0.0, y, 0.0)
        return carry

    lax.fori_loop(0, nb // L, rbody, 0)
    pltpu.sync_copy(dchunk_v, dinv_sh.at[pl.ds(s * nb, nb)])
    plsc.subcore_barrier()

    pltpu.sync_copy(dinv_sh, dinv_v)
    ebase = wid * _EPW
    pltpu.sync_copy(src_hbm.at[pl.ds(ebase, _EPW)], idx_v.at[pl.ds(0, _EPW)])
    pltpu.sync_copy(dst_hbm.at[pl.ds(ebase, _EPW)],
                    idx_v.at[pl.ds(_EPW, _EPW)])
    pltpu.sync_copy(attr_hbm.at[pl.ds(ebase, _EPW)],
                    attr_v.at[pl.ds(0, _EPW)])

    def nbody(i, carry):
        sv = idx_v[pl.ds(i * L, L)]
        dv = idx_v[pl.ds(_EPW + i * L, L)]
        av = attr_v[pl.ds(i * L, L)]
        nv = plsc.load_gather(dinv_v, [sv]) * plsc.load_gather(dinv_v, [dv])
        attr_v[pl.ds(i * L, L)] = nv * av
        return carry

    lax.fori_loop(0, _EPW // L, nbody, 0)
    pltpu.sync_copy(attr_v.at[pl.ds(0, _EPW)],
                    norm_hbm.at[pl.ds(ebase, _EPW)])


@functools.partial(
    pl.kernel,
    out_type=jax.ShapeDtypeStruct((NC, _ACCR, _D), jnp.float32),
    mesh=_mesh,
    compiler_params=_sc_params,
    scratch_types=[
        pltpu.VMEM((_EPAD,), jnp.int32),
        pltpu.VMEM((_EPAD,), jnp.int32),
        pltpu.VMEM((_EPAD,), jnp.float32),
        pltpu.VMEM((_G, _D), jnp.float32),
        pltpu.VMEM((_G, _D), jnp.float32),
        pltpu.VMEM((_G, _D), jnp.float32),
        pltpu.SemaphoreType.DMA,
        pltpu.SemaphoreType.DMA,
        pltpu.SemaphoreType.DMA,
        pltpu.SemaphoreType.DMA,
        pltpu.SemaphoreType.DMA,
        pltpu.SemaphoreType.DMA,
        pltpu.VMEM_SHARED((_ACCR, _D), jnp.float32),
    ],
)
def _sc_agg(h_hbm, src_hbm, dst_hbm, norm_hbm, part_hbm,
            src_v, dst_v, norm_v, rows0, rows1, rows2,
            gsem0, gsem1, gsem2, ssem0, ssem1, ssem2, acc_sh):
    c = lax.axis_index("c")
    s = lax.axis_index("s")
    wid = c * NS + s
    rpt = _ACCR // NS

    rows = (rows0, rows1, rows2)
    gsem = (gsem0, gsem1, gsem2)
    ssem = (ssem0, ssem1, ssem2)

    zero16 = jnp.zeros((L,), jnp.float32)

    def zb(i, carry):
        for q in range(_D // L):
            rows0[i, pl.ds(q * L, L)] = zero16
        return carry

    lax.fori_loop(0, _G, zb, 0)

    def zacc(j, carry):
        pltpu.sync_copy(rows0, acc_sh.at[pl.ds(s * rpt + j * _G, _G)])
        return carry

    lax.fori_loop(0, rpt // _G, zacc, 0)
    pltpu.sync_copy(rows0, acc_sh.at[pl.ds(s * rpt + rpt - _G, _G)])

    base = wid * _EPW
    pltpu.sync_copy(src_hbm.at[pl.ds(base, _EPW)], src_v.at[pl.ds(0, _EPW)])
    pltpu.sync_copy(dst_hbm.at[pl.ds(base, _EPW)], dst_v.at[pl.ds(0, _EPW)])
    pltpu.sync_copy(norm_hbm.at[pl.ds(base, _EPW)], norm_v.at[pl.ds(0, _EPW)])
    for t in range((_EPAD - _EPW) // L):
        src_v[pl.ds(_EPW + t * L, L)] = jnp.zeros((L,), jnp.int32)
        dst_v[pl.ds(_EPW + t * L, L)] = jnp.zeros((L,), jnp.int32) + _TRASH
        norm_v[pl.ds(_EPW + t * L, L)] = zero16

    plsc.subcore_barrier()

    def _gather(j, b):
        return pltpu.async_copy(
            h_hbm.at[src_v.at[pl.ds(j * _G, _G)]], rows[b], gsem[b]
        )

    def _scatter(j, b):
        return pltpu.async_copy(
            rows[b], acc_sh.at[dst_v.at[pl.ds(j * _G, _G)]], ssem[b], add=True
        )

    _gather(0, 0)

    def mbody(jo, carry):
        for b in range(_NBUF):
            j = jo * _NBUF + b
            bn = (b + 1) % _NBUF

            @pl.when(j >= 2)
            def _drain():
                pltpu.make_async_copy(
                    rows[bn], acc_sh.at[dst_v.at[pl.ds((j - 2) * _G, _G)]],
                    ssem[bn],
                ).wait()

            @pl.when(j + 1 < _NCHUNK)
            def _prefetch():
                _gather(j + 1, bn)

            pltpu.make_async_copy(
                h_hbm.at[src_v.at[pl.ds(j * _G, _G)]], rows[b], gsem[b]
            ).wait()

            def sbody(e, c2):
                idx = jnp.zeros((L,), jnp.int32) + (j * _G + e)
                nsc = plsc.load_gather(norm_v, [idx])
                for q in range(_D // L):
                    rows[b][e, pl.ds(q * L, L)] = (
                        rows[b][e, pl.ds(q * L, L)] * nsc
                    )
                return c2

            lax.fori_loop(0, _G, sbody, 0)
            _scatter(j, b)
        return carry

    lax.fori_loop(0, _NCHUNK // _NBUF, mbody, 0)

    for j in (_NCHUNK - 2, _NCHUNK - 1):
        b = j % _NBUF
        pltpu.make_async_copy(
            rows[b], acc_sh.at[dst_v.at[pl.ds(j * _G, _G)]], ssem[b]
        ).wait()

    plsc.subcore_barrier()
    pltpu.sync_copy(
        acc_sh.at[pl.ds(s * rpt, rpt)],
        part_hbm.at[c, pl.ds(s * rpt, rpt)],
    )


def _mm_body(x_ref, w_ref, o_ref):
    o_ref[...] = jnp.dot(
        x_ref[...], w_ref[...],
        preferred_element_type=jnp.float32,
        precision=lax.Precision.HIGHEST,
    )


def _comb_body(p0_ref, p1_ref, b_ref, o_ref):
    o_ref[...] = p0_ref[...] + p1_ref[...] + b_ref[...]


def kernel(x, edge_index, edge_attr, W, b):
    src = edge_index[0]
    dst = edge_index[1]

    h = pl.pallas_call(
        _mm_body,
        grid=(10,),
        in_specs=[
            pl.BlockSpec((_N // 10, _D), lambda i: (i, 0)),
            pl.BlockSpec((_D, _D), lambda i: (0, 0)),
        ],
        out_specs=pl.BlockSpec((_N // 10, _D), lambda i: (i, 0)),
        out_shape=jax.ShapeDtypeStruct((_N, _D), jnp.float32),
    )(x, W)

    norm = _sc_norm(src, dst, edge_attr)
    part = _sc_agg(h, src, dst, norm)

    out = pl.pallas_call(
        _comb_body,
        grid=(10,),
        in_specs=[
            pl.BlockSpec((_N // 10, _D), lambda i: (i, 0)),
            pl.BlockSpec((_N // 10, _D), lambda i: (i, 0)),
            pl.BlockSpec((1, _D), lambda i: (0, 0)),
        ],
        out_specs=pl.BlockSpec((_N // 10, _D), lambda i: (i, 0)),
        out_shape=jax.ShapeDtypeStruct((_N, _D), jnp.float32),
    )(part[0, :_N], part[1, :_N], b.reshape(1, _D))
    return out
